# baseline (speedup 1.0000x reference)
import functools

import numpy as np
import jax
import jax.numpy as jnp
from jax import lax
from jax.experimental import pallas as pl
from jax.experimental.pallas import tpu as pltpu
from jax.experimental.pallas import tpu_sc as plsc

MUL = 8
SQRT3 = 3.0 ** 0.5

NC, NS = 2, 16
NW = NC * NS
CS = 128
NP = 10240
E2 = 163840
PAD_DST = NP - 1

_S8 = np.repeat(np.eye(8, dtype=np.float32), 8, axis=1)
_T8 = np.tile(np.eye(8, dtype=np.float32), (1, 8))
_R64 = np.tile(np.eye(8, dtype=np.float32), (8, 1))
_SI512 = np.repeat(np.eye(8, dtype=np.float32), 64, axis=1)
_SJ512 = np.tile(np.repeat(np.eye(8, dtype=np.float32), 8, axis=1), (1, 8))
_SIJ512 = np.repeat(np.eye(64, dtype=np.float32), 8, axis=1)
_R512 = np.tile(np.eye(8, dtype=np.float32), (64, 1))
_P = np.zeros((3, 8, 24), dtype=np.float32)
for _a in range(3):
    for _k in range(8):
        _P[_a, _k, 3 * _k + _a] = 1.0

_Q24 = np.zeros((32, 24), dtype=np.float32)
for _a in range(3):
    _Q24[_a, _a * 8:(_a + 1) * 8] = 1.0
_P24 = np.zeros((32, 24), dtype=np.float32)
_P24[8:32, :] = np.eye(24, dtype=np.float32)
_RDS = np.tile(_S8, (3, 1))
_S64 = np.vstack([_S8, np.zeros((24, 64), np.float32)])
_CXYZ = []
for _a in range(3):
    _m = np.zeros((32, 64), dtype=np.float32)
    _m[8 + 8 * _a:16 + 8 * _a, :] = _S8
    _CXYZ.append(_m)
_T24 = np.tile(np.eye(8, dtype=np.float32), (1, 3))
_R64S = _R64 * 0.25
_RBT = (_R64 @ _T24) * (SQRT3 * 0.25)
_RC = []
for _a in range(3):
    _e = np.zeros((8, 24), dtype=np.float32)
    _e[:, _a * 8:(_a + 1) * 8] = np.eye(8, dtype=np.float32)
    _RC.append((_R64 @ _e) * 0.25)
_E8 = np.zeros((8, 32), dtype=np.float32)
_E8[:, 0:8] = np.eye(8, dtype=np.float32)
_E24 = np.zeros((24, 32), dtype=np.float32)
_E24[:, 8:32] = np.eye(24, dtype=np.float32)


def _silu(h):
    return h * (1.0 / (1.0 + jnp.exp(-h)))


def _dot(a, b):
    return jnp.dot(a, b, preferred_element_type=jnp.float32)


def _gather_body(xprep_hbm, posp_hbm, src2d_hbm, dst2d_hbm,
                 xg_hbm, ps_hbm, pd_hbm, sidx, didx, xbuf, psbuf, pdbuf, sem):
    c = lax.axis_index("c")
    s = lax.axis_index("s")
    w = s * NC + c
    ew = E2 // NW
    sub = ew // CS
    row0 = w * sub
    e0 = w * ew
    pltpu.sync_copy(src2d_hbm.at[pl.ds(row0, sub)], sidx)
    pltpu.sync_copy(dst2d_hbm.at[pl.ds(row0, sub)], didx)

    inner = 8
    rows = inner * CS

    def chunk(i, carry):
        copies = []
        for j in range(inner):
            r = i * inner + j
            copies.append(pltpu.async_copy(xprep_hbm.at[sidx.at[r]],
                                           xbuf.at[pl.ds(j * CS, CS)], sem))
            copies.append(pltpu.async_copy(posp_hbm.at[sidx.at[r]],
                                           psbuf.at[pl.ds(j * CS, CS)], sem))
            copies.append(pltpu.async_copy(posp_hbm.at[didx.at[r]],
                                           pdbuf.at[pl.ds(j * CS, CS)], sem))
        for cp in copies:
            cp.wait()
        off = e0 + i * rows
        pltpu.sync_copy(xbuf, xg_hbm.at[pl.ds(off, rows)])
        pltpu.sync_copy(psbuf, ps_hbm.at[pl.ds(off, rows)])
        pltpu.sync_copy(pdbuf, pd_hbm.at[pl.ds(off, rows)])
        return carry

    lax.fori_loop(0, sub // inner, chunk, 0)


def _gather_call(xprep, posp, src2d, dst2d):
    inner = 8
    rows = inner * CS
    sub = (E2 // NW) // CS
    mesh = plsc.VectorSubcoreMesh(core_axis_name="c", subcore_axis_name="s",
                                  num_cores=NC, num_subcores=NS)
    f = functools.partial(
        pl.kernel,
        out_type=(jax.ShapeDtypeStruct((E2, 32), jnp.float32),
                  jax.ShapeDtypeStruct((E2, 32), jnp.float32),
                  jax.ShapeDtypeStruct((E2, 32), jnp.float32)),
        mesh=mesh,
        scratch_types=[
            pltpu.VMEM((sub, CS), jnp.int32),
            pltpu.VMEM((sub, CS), jnp.int32),
            pltpu.VMEM((rows, 32), jnp.float32),
            pltpu.VMEM((rows, 32), jnp.float32),
            pltpu.VMEM((rows, 32), jnp.float32),
            pltpu.SemaphoreType.DMA,
        ],
        compiler_params=pltpu.CompilerParams(use_tc_tiling_on_sc=False),
    )(_gather_body)
    return f(xprep, posp, src2d, dst2d)


def _scatter_body(msg_hbm, dst2d_hbm, zeros_hbm, agg_hbm,
                  mbuf, dbuf, shared, sem):
    c = lax.axis_index("c")
    s = lax.axis_index("s")
    w = s * NC + c
    ew = E2 // NW
    sub = ew // CS
    row0 = w * sub
    e0 = w * ew
    npt = NP // NS

    pltpu.sync_copy(zeros_hbm.at[pl.ds(s * npt, npt)],
                    shared.at[pl.ds(s * npt, npt)])
    pltpu.sync_copy(dst2d_hbm.at[pl.ds(row0, sub)], dbuf)
    plsc.subcore_barrier()

    inner = 8
    rows = inner * CS

    def chunk(i, carry):
        off = e0 + i * rows
        pltpu.sync_copy(msg_hbm.at[pl.ds(off, rows)], mbuf)
        for j in range(inner):
            pltpu.sync_copy(mbuf.at[pl.ds(j * CS, CS)],
                            shared.at[dbuf.at[i * inner + j]], add=True)
        return carry

    lax.fori_loop(0, sub // inner, chunk, 0)
    plsc.subcore_barrier()
    pltpu.sync_copy(shared.at[pl.ds(s * npt, npt)],
                    agg_hbm.at[pl.ds(c * NP + s * npt, npt)])


def _scatter_call(msg, dst2d, zeros_pad):
    inner = 8
    rows = inner * CS
    sub = (E2 // NW) // CS
    mesh = plsc.VectorSubcoreMesh(core_axis_name="c", subcore_axis_name="s",
                                  num_cores=NC, num_subcores=NS)
    f = functools.partial(
        pl.kernel,
        out_type=jax.ShapeDtypeStruct((NC * NP, 32), jnp.float32),
        mesh=mesh,
        scratch_types=[
            pltpu.VMEM((rows, 32), jnp.float32),
            pltpu.VMEM((sub, CS), jnp.int32),
            pltpu.VMEM_SHARED((NP, 32), jnp.float32),
            pltpu.SemaphoreType.DMA,
        ],
        compiler_params=pltpu.CompilerParams(use_tc_tiling_on_sc=False),
    )(_scatter_body)
    return f(msg, dst2d, zeros_pad)


def _edge_body(ea_ref, xg_ref, ps_ref, pd_ref, w1_ref, w2_ref,
               q24_ref, p24_ref, rds_ref, s64_ref, cx_ref, cy_ref, cz_ref,
               r64s_ref, rbt_ref, rcx_ref, rcy_ref, rcz_ref, e8_ref, e24_ref,
               out_ref):
    Q = xg_ref.shape[0]
    for e in range(4):
        ea = jnp.transpose(ea_ref[:, e * Q:(e + 1) * Q], (1, 0))
        h = _dot(ea, w1_ref[...])
        w = _dot(_silu(h), w2_ref[...])
        wA = w[:, 0:64]
        wB = w[:, 64:128]
        wC = w[:, 128:192]
        wD = w[:, 192:256]

        d = (pd_ref[:, e * 32:(e + 1) * 32]
             - ps_ref[:, e * 32:(e + 1) * 32])
        rsq = jnp.sum(d * d, axis=1, keepdims=True)
        rinv = 1.0 / jnp.sqrt(rsq + 1e-12)
        u16 = d * rinv

        xg = xg_ref[:, e * 32:(e + 1) * 32]
        U24 = _dot(u16, q24_ref[...])
        dvmul = _dot(xg, p24_ref[...]) * U24
        dvr = _dot(dvmul, rds_ref[...])
        xsr = _dot(xg, s64_ref[...])
        out_s8 = _dot(xsr * wA + dvr * wD, r64s_ref[...])
        tb24 = _dot(xsr * wB, rbt_ref[...])
        tc24 = (_dot(_dot(xg, cx_ref[...]) * wC, rcx_ref[...])
                + _dot(_dot(xg, cy_ref[...]) * wC, rcy_ref[...])
                + _dot(_dot(xg, cz_ref[...]) * wC, rcz_ref[...]))
        ov24 = U24 * tb24 + tc24
        out32 = _dot(out_s8, e8_ref[...]) + _dot(ov24, e24_ref[...])
        out_ref[:, e * 32:(e + 1) * 32] = out32


def _edge_call(edge_attr, xg4, ps8, pd8, W1, W2, E, interpret=False):
    BE = 3200
    grid = (E // BE,)
    full = lambda s: pl.BlockSpec(s, lambda i: (0, 0))
    return pl.pallas_call(
        _edge_body,
        grid=grid,
        in_specs=[
            pl.BlockSpec((16, BE), lambda i: (0, i)),
            pl.BlockSpec((BE // 4, 128), lambda i: (i, 0)),
            pl.BlockSpec((BE // 4, 128), lambda i: (i, 0)),
            pl.BlockSpec((BE // 4, 128), lambda i: (i, 0)),
            full((16, 64)),
            full((64, 256)),
            full((32, 24)),
            full((32, 24)),
            full((24, 64)),
            full((32, 64)),
            full((32, 64)),
            full((32, 64)),
            full((32, 64)),
            full((64, 8)),
            full((64, 24)),
            full((64, 24)),
            full((64, 24)),
            full((64, 24)),
            full((8, 32)),
            full((24, 32)),
        ],
        out_specs=pl.BlockSpec((BE // 4, 128), lambda i: (i, 0)),
        out_shape=jax.ShapeDtypeStruct((E2 // 4, 128), jnp.float32),
        interpret=interpret,
    )(edge_attr.T, xg4, ps8, pd8, W1 * 0.25, W2 * 0.125,
      jnp.asarray(_Q24), jnp.asarray(_P24), jnp.asarray(_RDS),
      jnp.asarray(_S64), jnp.asarray(_CXYZ[0]), jnp.asarray(_CXYZ[1]),
      jnp.asarray(_CXYZ[2]), jnp.asarray(_R64S), jnp.asarray(_RBT),
      jnp.asarray(_RC[0]), jnp.asarray(_RC[1]), jnp.asarray(_RC[2]),
      jnp.asarray(_E8), jnp.asarray(_E24))


def _node_body(inv_sqrt_deg, xp_ref, a0_ref, a1_ref, oh_ref, u1_ref, u2_ref,
               s8_ref, t8_ref, si_ref, sj_ref, sij_ref, r512_ref,
               px_ref, py_ref, pz_ref, out_ref):
    h = _dot(oh_ref[...], u1_ref[...])
    w = _dot(_silu(h), u2_ref[...])
    wA = w[:, 0:512]
    wB = w[:, 512:1024]
    wC = w[:, 1024:1536]
    wD = w[:, 1536:2048]

    agg = (a0_ref[...] + a1_ref[...]) * inv_sqrt_deg
    ms = agg[:, 0:8]
    mvx = agg[:, 8:16]
    mvy = agg[:, 16:24]
    mvz = agg[:, 24:32]
    xp = xp_ref[...]
    xs = xp[:, 0:8]
    xvx = xp[:, 8:16]
    xvy = xp[:, 16:24]
    xvz = xp[:, 24:32]

    S8 = s8_ref[...]
    T8 = t8_ref[...]
    SI = si_ref[...]
    SJ = sj_ref[...]
    R512 = r512_ref[...]
    fan_inv = 1.0 / np.sqrt(2.0 * 64.0)

    dvv = (_dot(xvx, S8) * _dot(mvx, T8)
           + _dot(xvy, S8) * _dot(mvy, T8)
           + _dot(xvz, S8) * _dot(mvz, T8))
    XS = _dot(xs, SI)
    MS = _dot(ms, SJ)
    DV = _dot(dvv, sij_ref[...])

    out_s = _dot(XS * MS * wA + DV * wD * (1.0 / SQRT3), R512) * fan_inv
    ovx = _dot(XS * _dot(mvx, SJ) * wB + _dot(xvx, SI) * MS * wC, R512) * fan_inv
    ovy = _dot(XS * _dot(mvy, SJ) * wB + _dot(xvy, SI) * MS * wC, R512) * fan_inv
    ovz = _dot(XS * _dot(mvz, SJ) * wB + _dot(xvz, SI) * MS * wC, R512) * fan_inv
    out_v = (_dot(ovx, px_ref[...]) + _dot(ovy, py_ref[...])
             + _dot(ovz, pz_ref[...]))
    out_ref[...] = jnp.concatenate([out_s, out_v], axis=1)


def _node_call(xprep, a0, a1, onehot, U1, U2, inv_sqrt_deg, N, interpret=False):
    BN = 1000
    grid = (N // BN,)
    return pl.pallas_call(
        functools.partial(_node_body, inv_sqrt_deg),
        grid=grid,
        in_specs=[
            pl.BlockSpec((BN, 32), lambda i: (i, 0)),
            pl.BlockSpec((BN, 32), lambda i: (i, 0)),
            pl.BlockSpec((BN, 32), lambda i: (i, 0)),
            pl.BlockSpec((BN, 16), lambda i: (i, 0)),
            pl.BlockSpec((16, 64), lambda i: (0, 0)),
            pl.BlockSpec((64, 2048), lambda i: (0, 0)),
            pl.BlockSpec((8, 64), lambda i: (0, 0)),
            pl.BlockSpec((8, 64), lambda i: (0, 0)),
            pl.BlockSpec((8, 512), lambda i: (0, 0)),
            pl.BlockSpec((8, 512), lambda i: (0, 0)),
            pl.BlockSpec((64, 512), lambda i: (0, 0)),
            pl.BlockSpec((512, 8), lambda i: (0, 0)),
            pl.BlockSpec((8, 24), lambda i: (0, 0)),
            pl.BlockSpec((8, 24), lambda i: (0, 0)),
            pl.BlockSpec((8, 24), lambda i: (0, 0)),
        ],
        out_specs=pl.BlockSpec((BN, 32), lambda i: (i, 0)),
        out_shape=jax.ShapeDtypeStruct((N, 32), jnp.float32),
        interpret=interpret,
    )(xprep, a0, a1, onehot, U1 * 0.25, U2 * 0.125,
      jnp.asarray(_S8), jnp.asarray(_T8), jnp.asarray(_SI512),
      jnp.asarray(_SJ512), jnp.asarray(_SIJ512), jnp.asarray(_R512),
      jnp.asarray(_P[0]), jnp.asarray(_P[1]), jnp.asarray(_P[2]))


def _prep(x, pos):
    N = x.shape[0]
    xv = x[:, MUL:].reshape(N, MUL, 3)
    xprep = jnp.concatenate([x[:, :MUL], xv[:, :, 0], xv[:, :, 1], xv[:, :, 2]],
                            axis=1)
    posp = jnp.concatenate([pos, jnp.zeros((N, 29), pos.dtype)], axis=1)
    return xprep, posp


def kernel(x, pos, edge_attr, onehot_node_features, W1, W2, U1, U2, edge_index):
    N = x.shape[0]
    E = edge_index.shape[1]
    xprep, posp = _prep(x, pos)
    BE = 3200

    def perm4(a):
        return a.reshape(E // BE, 4, BE // 4).transpose(0, 2, 1).reshape(E)

    src_p = jnp.concatenate([perm4(edge_index[0]),
                             jnp.zeros((E2 - E,), jnp.int32)])
    dst_p = jnp.concatenate([perm4(edge_index[1]),
                             jnp.full((E2 - E,), PAD_DST, jnp.int32)])
    src2d = src_p.reshape(E2 // CS, CS)
    dst2d = dst_p.reshape(E2 // CS, CS)
    zeros_pad = jnp.zeros((NP, 32), jnp.float32)

    xg, ps, pd = _gather_call(xprep, posp, src2d, dst2d)
    msg4 = _edge_call(edge_attr, xg.reshape(E2 // 4, 128),
                      ps.reshape(E2 // 4, 128), pd.reshape(E2 // 4, 128),
                      W1, W2, E)
    agg = _scatter_call(msg4.reshape(E2, 32), dst2d, zeros_pad)
    inv_sqrt_deg = float(1.0 / np.sqrt(E / N))
    return _node_call(xprep, agg[:N], agg[NP:NP + N], onehot_node_features,
                      U1, U2, inv_sqrt_deg, N)


# speedup vs baseline: 1.0800x; 1.0800x over previous
import functools

import numpy as np
import jax
import jax.numpy as jnp
from jax import lax
from jax.experimental import pallas as pl
from jax.experimental.pallas import tpu as pltpu
from jax.experimental.pallas import tpu_sc as plsc

MUL = 8
SQRT3 = 3.0 ** 0.5

NC, NS = 2, 16
NW = NC * NS
CS = 125
NP = 10240

_S8 = np.repeat(np.eye(8, dtype=np.float32), 8, axis=1)
_T8 = np.tile(np.eye(8, dtype=np.float32), (1, 8))
_R64 = np.tile(np.eye(8, dtype=np.float32), (8, 1))
_SI512 = np.repeat(np.eye(8, dtype=np.float32), 64, axis=1)
_SJ512 = np.tile(np.repeat(np.eye(8, dtype=np.float32), 8, axis=1), (1, 8))
_SIJ512 = np.repeat(np.eye(64, dtype=np.float32), 8, axis=1)
_R512 = np.tile(np.eye(8, dtype=np.float32), (64, 1))
_P = np.zeros((3, 8, 24), dtype=np.float32)
for _a in range(3):
    for _k in range(8):
        _P[_a, _k, 3 * _k + _a] = 1.0

_Q24 = np.zeros((16, 24), dtype=np.float32)
for _a in range(3):
    _Q24[_a, _a * 8:(_a + 1) * 8] = 1.0
_P24 = np.zeros((32, 24), dtype=np.float32)
_P24[8:32, :] = np.eye(24, dtype=np.float32)
_RDS = np.tile(_S8, (3, 1))
_S64 = np.vstack([_S8, np.zeros((24, 64), np.float32)])
_CXYZ = []
for _a in range(3):
    _m = np.zeros((32, 64), dtype=np.float32)
    _m[8 + 8 * _a:16 + 8 * _a, :] = _S8
    _CXYZ.append(_m)
_T24 = np.tile(np.eye(8, dtype=np.float32), (1, 3))
_R64S = _R64 * 0.25
_RBT = (_R64 @ _T24) * (SQRT3 * 0.25)
_RC = []
for _a in range(3):
    _e = np.zeros((8, 24), dtype=np.float32)
    _e[:, _a * 8:(_a + 1) * 8] = np.eye(8, dtype=np.float32)
    _RC.append((_R64 @ _e) * 0.25)
_E8 = np.zeros((8, 32), dtype=np.float32)
_E8[:, 0:8] = np.eye(8, dtype=np.float32)
_E24 = np.zeros((24, 32), dtype=np.float32)
_E24[:, 8:32] = np.eye(24, dtype=np.float32)


def _silu(h):
    return h * (1.0 / (1.0 + jnp.exp(-h)))


def _dot(a, b):
    return jnp.dot(a, b, preferred_element_type=jnp.float32)


def _gather_body(E, xprep_hbm, posp_hbm, src2d_hbm, dst2d_hbm,
                 xg_hbm, ps_hbm, pd_hbm, sidx, didx, xbuf, psbuf, pdbuf, sem):
    c = lax.axis_index("c")
    s = lax.axis_index("s")
    w = s * NC + c
    ew = E // NW
    sub = ew // CS
    row0 = w * sub
    e0 = w * ew
    pltpu.sync_copy(src2d_hbm.at[pl.ds(row0, sub)], sidx)
    pltpu.sync_copy(dst2d_hbm.at[pl.ds(row0, sub)], didx)

    inner = 8
    rows = inner * CS

    def chunk(i, carry):
        copies = []
        for j in range(inner):
            r = i * inner + j
            copies.append(pltpu.async_copy(xprep_hbm.at[sidx.at[r]],
                                           xbuf.at[pl.ds(j * CS, CS)], sem))
            copies.append(pltpu.async_copy(posp_hbm.at[sidx.at[r]],
                                           psbuf.at[pl.ds(j * CS, CS)], sem))
            copies.append(pltpu.async_copy(posp_hbm.at[didx.at[r]],
                                           pdbuf.at[pl.ds(j * CS, CS)], sem))
        for cp in copies:
            cp.wait()
        off = e0 + i * rows
        pltpu.sync_copy(xbuf, xg_hbm.at[pl.ds(off, rows)])
        pltpu.sync_copy(psbuf, ps_hbm.at[pl.ds(off, rows)])
        pltpu.sync_copy(pdbuf, pd_hbm.at[pl.ds(off, rows)])
        return carry

    lax.fori_loop(0, sub // inner, chunk, 0)


def _gather_call(xprep, posp, src2d, dst2d, E):
    inner = 8
    rows = inner * CS
    sub = (E // NW) // CS
    mesh = plsc.VectorSubcoreMesh(core_axis_name="c", subcore_axis_name="s",
                                  num_cores=NC, num_subcores=NS)
    f = functools.partial(
        pl.kernel,
        out_type=(jax.ShapeDtypeStruct((E, 32), jnp.float32),
                  jax.ShapeDtypeStruct((E, 16), jnp.float32),
                  jax.ShapeDtypeStruct((E, 16), jnp.float32)),
        mesh=mesh,
        scratch_types=[
            pltpu.VMEM((sub, CS), jnp.int32),
            pltpu.VMEM((sub, CS), jnp.int32),
            pltpu.VMEM((rows, 32), jnp.float32),
            pltpu.VMEM((rows, 16), jnp.float32),
            pltpu.VMEM((rows, 16), jnp.float32),
            pltpu.SemaphoreType.DMA,
        ],
        compiler_params=pltpu.CompilerParams(use_tc_tiling_on_sc=False),
    )(functools.partial(_gather_body, E))
    return f(xprep, posp, src2d, dst2d)


def _scatter_body(E, msg_hbm, dst2d_hbm, zeros_hbm, agg_hbm,
                  mbuf, dbuf, shared, sem):
    c = lax.axis_index("c")
    s = lax.axis_index("s")
    w = s * NC + c
    ew = E // NW
    sub = ew // CS
    row0 = w * sub
    e0 = w * ew
    npt = NP // NS

    pltpu.sync_copy(zeros_hbm.at[pl.ds(s * npt, npt)],
                    shared.at[pl.ds(s * npt, npt)])
    pltpu.sync_copy(dst2d_hbm.at[pl.ds(row0, sub)], dbuf)
    plsc.subcore_barrier()

    inner = 8
    rows = inner * CS

    def chunk(i, carry):
        off = e0 + i * rows
        pltpu.sync_copy(msg_hbm.at[pl.ds(off, rows)], mbuf)
        for j in range(inner):
            pltpu.sync_copy(mbuf.at[pl.ds(j * CS, CS)],
                            shared.at[dbuf.at[i * inner + j]], add=True)
        return carry

    lax.fori_loop(0, sub // inner, chunk, 0)
    plsc.subcore_barrier()
    pltpu.sync_copy(shared.at[pl.ds(s * npt, npt)],
                    agg_hbm.at[pl.ds(c * NP + s * npt, npt)])


def _scatter_call(msg, dst2d, zeros_pad, E):
    inner = 8
    rows = inner * CS
    sub = (E // NW) // CS
    mesh = plsc.VectorSubcoreMesh(core_axis_name="c", subcore_axis_name="s",
                                  num_cores=NC, num_subcores=NS)
    f = functools.partial(
        pl.kernel,
        out_type=jax.ShapeDtypeStruct((NC * NP, 32), jnp.float32),
        mesh=mesh,
        scratch_types=[
            pltpu.VMEM((rows, 32), jnp.float32),
            pltpu.VMEM((sub, CS), jnp.int32),
            pltpu.VMEM_SHARED((NP, 32), jnp.float32),
            pltpu.SemaphoreType.DMA,
        ],
        compiler_params=pltpu.CompilerParams(use_tc_tiling_on_sc=False),
    )(functools.partial(_scatter_body, E))
    return f(msg, dst2d, zeros_pad)


def _edge_body(ea_ref, xg_ref, ps_ref, pd_ref, w1_ref, w2_ref,
               q24_ref, p24_ref, rds_ref, s64_ref, cx_ref, cy_ref, cz_ref,
               r64s_ref, rbt_ref, rcx_ref, rcy_ref, rcz_ref, e8_ref, e24_ref,
               out_ref):
    ea = jnp.transpose(ea_ref[...], (1, 0))
    h = _dot(ea, w1_ref[...])
    w = _dot(_silu(h), w2_ref[...])
    wA = w[:, 0:64]
    wB = w[:, 64:128]
    wC = w[:, 128:192]
    wD = w[:, 192:256]

    d = pd_ref[...] - ps_ref[...]
    rsq = jnp.sum(d * d, axis=1, keepdims=True)
    rinv = 1.0 / jnp.sqrt(rsq + 1e-12)
    u16 = d * rinv

    xg = xg_ref[...]
    U24 = _dot(u16, q24_ref[...])
    dvmul = _dot(xg, p24_ref[...]) * U24
    dvr = _dot(dvmul, rds_ref[...])
    xsr = _dot(xg, s64_ref[...])
    out_s8 = _dot(xsr * wA + dvr * wD, r64s_ref[...])
    tb24 = _dot(xsr * wB, rbt_ref[...])
    tc24 = (_dot(_dot(xg, cx_ref[...]) * wC, rcx_ref[...])
            + _dot(_dot(xg, cy_ref[...]) * wC, rcy_ref[...])
            + _dot(_dot(xg, cz_ref[...]) * wC, rcz_ref[...]))
    ov24 = U24 * tb24 + tc24
    out_ref[...] = _dot(out_s8, e8_ref[...]) + _dot(ov24, e24_ref[...])


def _edge_call(edge_attr, xg, ps, pd, W1, W2, E, interpret=False):
    BE = 6400
    grid = (E // BE,)
    full = lambda s: pl.BlockSpec(s, lambda i: (0, 0))
    return pl.pallas_call(
        _edge_body,
        grid=grid,
        in_specs=[
            pl.BlockSpec((16, BE), lambda i: (0, i)),
            pl.BlockSpec((BE, 32), lambda i: (i, 0)),
            pl.BlockSpec((BE, 16), lambda i: (i, 0)),
            pl.BlockSpec((BE, 16), lambda i: (i, 0)),
            full((16, 64)),
            full((64, 256)),
            full((16, 24)),
            full((32, 24)),
            full((24, 64)),
            full((32, 64)),
            full((32, 64)),
            full((32, 64)),
            full((32, 64)),
            full((64, 8)),
            full((64, 24)),
            full((64, 24)),
            full((64, 24)),
            full((64, 24)),
            full((8, 32)),
            full((24, 32)),
        ],
        out_specs=pl.BlockSpec((BE, 32), lambda i: (i, 0)),
        out_shape=jax.ShapeDtypeStruct((E, 32), jnp.float32),
        interpret=interpret,
    )(edge_attr.T, xg, ps, pd, W1 * 0.25, W2 * 0.125,
      jnp.asarray(_Q24), jnp.asarray(_P24), jnp.asarray(_RDS),
      jnp.asarray(_S64), jnp.asarray(_CXYZ[0]), jnp.asarray(_CXYZ[1]),
      jnp.asarray(_CXYZ[2]), jnp.asarray(_R64S), jnp.asarray(_RBT),
      jnp.asarray(_RC[0]), jnp.asarray(_RC[1]), jnp.asarray(_RC[2]),
      jnp.asarray(_E8), jnp.asarray(_E24))


def _node_body(inv_sqrt_deg, xp_ref, a0_ref, a1_ref, oh_ref, u1_ref, u2_ref,
               s8_ref, t8_ref, si_ref, sj_ref, sij_ref, r512_ref,
               px_ref, py_ref, pz_ref, out_ref):
    h = _dot(oh_ref[...], u1_ref[...])
    w = _dot(_silu(h), u2_ref[...])
    wA = w[:, 0:512]
    wB = w[:, 512:1024]
    wC = w[:, 1024:1536]
    wD = w[:, 1536:2048]

    agg = (a0_ref[...] + a1_ref[...]) * inv_sqrt_deg
    ms = agg[:, 0:8]
    mvx = agg[:, 8:16]
    mvy = agg[:, 16:24]
    mvz = agg[:, 24:32]
    xp = xp_ref[...]
    xs = xp[:, 0:8]
    xvx = xp[:, 8:16]
    xvy = xp[:, 16:24]
    xvz = xp[:, 24:32]

    S8 = s8_ref[...]
    T8 = t8_ref[...]
    SI = si_ref[...]
    SJ = sj_ref[...]
    R512 = r512_ref[...]
    fan_inv = 1.0 / np.sqrt(2.0 * 64.0)

    dvv = (_dot(xvx, S8) * _dot(mvx, T8)
           + _dot(xvy, S8) * _dot(mvy, T8)
           + _dot(xvz, S8) * _dot(mvz, T8))
    XS = _dot(xs, SI)
    MS = _dot(ms, SJ)
    DV = _dot(dvv, sij_ref[...])

    out_s = _dot(XS * MS * wA + DV * wD * (1.0 / SQRT3), R512) * fan_inv
    ovx = _dot(XS * _dot(mvx, SJ) * wB + _dot(xvx, SI) * MS * wC, R512) * fan_inv
    ovy = _dot(XS * _dot(mvy, SJ) * wB + _dot(xvy, SI) * MS * wC, R512) * fan_inv
    ovz = _dot(XS * _dot(mvz, SJ) * wB + _dot(xvz, SI) * MS * wC, R512) * fan_inv
    out_v = (_dot(ovx, px_ref[...]) + _dot(ovy, py_ref[...])
             + _dot(ovz, pz_ref[...]))
    out_ref[...] = jnp.concatenate([out_s, out_v], axis=1)


def _node_call(xprep, a0, a1, onehot, U1, U2, inv_sqrt_deg, N, interpret=False):
    BN = 1000
    grid = (N // BN,)
    return pl.pallas_call(
        functools.partial(_node_body, inv_sqrt_deg),
        grid=grid,
        in_specs=[
            pl.BlockSpec((BN, 32), lambda i: (i, 0)),
            pl.BlockSpec((BN, 32), lambda i: (i, 0)),
            pl.BlockSpec((BN, 32), lambda i: (i, 0)),
            pl.BlockSpec((BN, 16), lambda i: (i, 0)),
            pl.BlockSpec((16, 64), lambda i: (0, 0)),
            pl.BlockSpec((64, 2048), lambda i: (0, 0)),
            pl.BlockSpec((8, 64), lambda i: (0, 0)),
            pl.BlockSpec((8, 64), lambda i: (0, 0)),
            pl.BlockSpec((8, 512), lambda i: (0, 0)),
            pl.BlockSpec((8, 512), lambda i: (0, 0)),
            pl.BlockSpec((64, 512), lambda i: (0, 0)),
            pl.BlockSpec((512, 8), lambda i: (0, 0)),
            pl.BlockSpec((8, 24), lambda i: (0, 0)),
            pl.BlockSpec((8, 24), lambda i: (0, 0)),
            pl.BlockSpec((8, 24), lambda i: (0, 0)),
        ],
        out_specs=pl.BlockSpec((BN, 32), lambda i: (i, 0)),
        out_shape=jax.ShapeDtypeStruct((N, 32), jnp.float32),
        interpret=interpret,
    )(xprep, a0, a1, onehot, U1 * 0.25, U2 * 0.125,
      jnp.asarray(_S8), jnp.asarray(_T8), jnp.asarray(_SI512),
      jnp.asarray(_SJ512), jnp.asarray(_SIJ512), jnp.asarray(_R512),
      jnp.asarray(_P[0]), jnp.asarray(_P[1]), jnp.asarray(_P[2]))


def _prep(x, pos):
    N = x.shape[0]
    xv = x[:, MUL:].reshape(N, MUL, 3)
    xprep = jnp.concatenate([x[:, :MUL], xv[:, :, 0], xv[:, :, 1], xv[:, :, 2]],
                            axis=1)
    posp = jnp.concatenate([pos, jnp.zeros((N, 13), pos.dtype)], axis=1)
    return xprep, posp


def kernel(x, pos, edge_attr, onehot_node_features, W1, W2, U1, U2, edge_index):
    N = x.shape[0]
    E = edge_index.shape[1]
    xprep, posp = _prep(x, pos)
    src2d = edge_index[0].reshape(E // CS, CS)
    dst2d = edge_index[1].reshape(E // CS, CS)
    zeros_pad = jnp.zeros((NP, 32), jnp.float32)

    xg, ps, pd = _gather_call(xprep, posp, src2d, dst2d, E)
    msg = _edge_call(edge_attr, xg, ps, pd, W1, W2, E)
    agg = _scatter_call(msg, dst2d, zeros_pad, E)
    inv_sqrt_deg = float(1.0 / np.sqrt(E / N))
    return _node_call(xprep, agg[:N], agg[NP:NP + N], onehot_node_features,
                      U1, U2, inv_sqrt_deg, N)


# speedup vs baseline: 1.1255x; 1.0422x over previous
import functools

import numpy as np
import jax
import jax.numpy as jnp
from jax import lax
from jax.experimental import pallas as pl
from jax.experimental.pallas import tpu as pltpu
from jax.experimental.pallas import tpu_sc as plsc

MUL = 8
SQRT3 = 3.0 ** 0.5

NC, NS = 2, 16
NW = NC * NS
CS = 125
NP = 10240

_S8 = np.repeat(np.eye(8, dtype=np.float32), 8, axis=1)
_T8 = np.tile(np.eye(8, dtype=np.float32), (1, 8))
_R64 = np.tile(np.eye(8, dtype=np.float32), (8, 1))
_SI512 = np.repeat(np.eye(8, dtype=np.float32), 64, axis=1)
_SJ512 = np.tile(np.repeat(np.eye(8, dtype=np.float32), 8, axis=1), (1, 8))
_SIJ512 = np.repeat(np.eye(64, dtype=np.float32), 8, axis=1)
_R512 = np.tile(np.eye(8, dtype=np.float32), (64, 1))
_P = np.zeros((3, 8, 24), dtype=np.float32)
for _a in range(3):
    for _k in range(8):
        _P[_a, _k, 3 * _k + _a] = 1.0

_Q24 = np.zeros((16, 24), dtype=np.float32)
for _a in range(3):
    _Q24[_a, _a * 8:(_a + 1) * 8] = 1.0
_P24 = np.zeros((32, 24), dtype=np.float32)
_P24[8:32, :] = np.eye(24, dtype=np.float32)
_RDS = np.tile(_S8, (3, 1))
_S64 = np.vstack([_S8, np.zeros((24, 64), np.float32)])
_CXYZ = []
for _a in range(3):
    _m = np.zeros((32, 64), dtype=np.float32)
    _m[8 + 8 * _a:16 + 8 * _a, :] = _S8
    _CXYZ.append(_m)
_T24 = np.tile(np.eye(8, dtype=np.float32), (1, 3))
_R64S = _R64 * 0.25
_RBT = (_R64 @ _T24) * (SQRT3 * 0.25)
_RC = []
for _a in range(3):
    _e = np.zeros((8, 24), dtype=np.float32)
    _e[:, _a * 8:(_a + 1) * 8] = np.eye(8, dtype=np.float32)
    _RC.append((_R64 @ _e) * 0.25)
_E8 = np.zeros((8, 32), dtype=np.float32)
_E8[:, 0:8] = np.eye(8, dtype=np.float32)
_E24 = np.zeros((24, 32), dtype=np.float32)
_E24[:, 8:32] = np.eye(24, dtype=np.float32)


def _silu(h):
    return h * (1.0 / (1.0 + jnp.exp(-h)))


def _dot(a, b):
    return jnp.dot(a, b, preferred_element_type=jnp.float32)


def _gather_body(E, table_hbm, idx2d_hbm, g_hbm, gidx, gbuf, sem):
    c = lax.axis_index("c")
    s = lax.axis_index("s")
    w = s * NC + c
    ew = 2 * E // NW
    sub = ew // CS
    row0 = w * sub
    e0 = w * ew
    pltpu.sync_copy(idx2d_hbm.at[pl.ds(row0, sub)], gidx)

    inner = 8
    rows = inner * CS

    def chunk(i, carry):
        copies = []
        for j in range(inner):
            r = i * inner + j
            copies.append(pltpu.async_copy(table_hbm.at[gidx.at[r]],
                                           gbuf.at[pl.ds(j * CS, CS)], sem))
        for cp in copies:
            cp.wait()
        pltpu.sync_copy(gbuf, g_hbm.at[pl.ds(e0 + i * rows, rows)])
        return carry

    lax.fori_loop(0, sub // inner, chunk, 0)


def _gather_call(table, idx2d, E):
    inner = 8
    rows = inner * CS
    sub = (2 * E // NW) // CS
    mesh = plsc.VectorSubcoreMesh(core_axis_name="c", subcore_axis_name="s",
                                  num_cores=NC, num_subcores=NS)
    f = functools.partial(
        pl.kernel,
        out_type=jax.ShapeDtypeStruct((2 * E, 64), jnp.float32),
        mesh=mesh,
        scratch_types=[
            pltpu.VMEM((sub, CS), jnp.int32),
            pltpu.VMEM((rows, 64), jnp.float32),
            pltpu.SemaphoreType.DMA,
        ],
        compiler_params=pltpu.CompilerParams(use_tc_tiling_on_sc=False),
    )(functools.partial(_gather_body, E))
    return f(table, idx2d)


def _scatter_body(E, msg_hbm, dst2d_hbm, zeros_hbm, agg_hbm,
                  mbuf, dbuf, shared, sem):
    c = lax.axis_index("c")
    s = lax.axis_index("s")
    w = s * NC + c
    ew = E // NW
    sub = ew // CS
    row0 = w * sub
    e0 = w * ew
    npt = NP // NS

    pltpu.sync_copy(zeros_hbm.at[pl.ds(s * npt, npt)],
                    shared.at[pl.ds(s * npt, npt)])
    pltpu.sync_copy(dst2d_hbm.at[pl.ds(row0, sub)], dbuf)
    plsc.subcore_barrier()

    inner = 8
    rows = inner * CS

    def chunk(i, carry):
        off = e0 + i * rows
        pltpu.sync_copy(msg_hbm.at[pl.ds(off, rows)], mbuf)
        for j in range(inner):
            pltpu.sync_copy(mbuf.at[pl.ds(j * CS, CS)],
                            shared.at[dbuf.at[i * inner + j]], add=True)
        return carry

    lax.fori_loop(0, sub // inner, chunk, 0)
    plsc.subcore_barrier()
    pltpu.sync_copy(shared.at[pl.ds(s * npt, npt)],
                    agg_hbm.at[pl.ds(c * NP + s * npt, npt)])


def _scatter_call(msg, dst2d, zeros_pad, E):
    inner = 8
    rows = inner * CS
    sub = (E // NW) // CS
    mesh = plsc.VectorSubcoreMesh(core_axis_name="c", subcore_axis_name="s",
                                  num_cores=NC, num_subcores=NS)
    f = functools.partial(
        pl.kernel,
        out_type=jax.ShapeDtypeStruct((NC * NP, 32), jnp.float32),
        mesh=mesh,
        scratch_types=[
            pltpu.VMEM((rows, 32), jnp.float32),
            pltpu.VMEM((sub, CS), jnp.int32),
            pltpu.VMEM_SHARED((NP, 32), jnp.float32),
            pltpu.SemaphoreType.DMA,
        ],
        compiler_params=pltpu.CompilerParams(use_tc_tiling_on_sc=False),
    )(functools.partial(_scatter_body, E))
    return f(msg, dst2d, zeros_pad)


def _edge_body(ea_ref, g_ref, w1_ref, w2_ref, dif_ref,
               q24_ref, p24_ref, rds_ref, s64_ref, cx_ref, cy_ref, cz_ref,
               r64s_ref, rbt_ref, rcx_ref, rcy_ref, rcz_ref, e8_ref, e24_ref,
               out_ref):
    ea = jnp.transpose(ea_ref[...], (1, 0))
    h = _dot(ea, w1_ref[...])
    w = _dot(_silu(h), w2_ref[...])
    wA = w[:, 0:64]
    wB = w[:, 64:128]
    wC = w[:, 128:192]
    wD = w[:, 192:256]

    g = g_ref[...]
    d = _dot(g, dif_ref[...])
    rsq = jnp.sum(d * d, axis=1, keepdims=True)
    rinv = 1.0 / jnp.sqrt(rsq + 1e-12)
    u16 = d * rinv

    U24 = _dot(u16, q24_ref[...])
    dvmul = _dot(g, p24_ref[...]) * U24
    dvr = _dot(dvmul, rds_ref[...])
    xsr = _dot(g, s64_ref[...])
    out_s8 = _dot(xsr * wA + dvr * wD, r64s_ref[...])
    tb24 = _dot(xsr * wB, rbt_ref[...])
    tc24 = (_dot(_dot(g, cx_ref[...]) * wC, rcx_ref[...])
            + _dot(_dot(g, cy_ref[...]) * wC, rcy_ref[...])
            + _dot(_dot(g, cz_ref[...]) * wC, rcz_ref[...]))
    ov24 = U24 * tb24 + tc24
    out_ref[:, 0:8] = out_s8
    out_ref[:, 8:32] = ov24


def _edge_call(edge_attr, g, W1, W2, E, interpret=False):
    BE = 6400
    grid = (E // BE,)
    full = lambda s: pl.BlockSpec(s, lambda i: (0, 0))
    pad128 = lambda m: np.vstack([m, np.zeros((128 - m.shape[0], m.shape[1]),
                                              np.float32)])
    dif = np.zeros((128, 16), np.float32)
    dif[np.arange(32, 48), np.arange(16)] = -1.0
    dif[np.arange(96, 112), np.arange(16)] = 1.0
    return pl.pallas_call(
        _edge_body,
        grid=grid,
        in_specs=[
            pl.BlockSpec((16, BE), lambda i: (0, i)),
            pl.BlockSpec((BE, 128), lambda i: (i, 0)),
            full((16, 64)),
            full((64, 256)),
            full((128, 16)),
            full((16, 24)),
            full((128, 24)),
            full((24, 64)),
            full((128, 64)),
            full((128, 64)),
            full((128, 64)),
            full((128, 64)),
            full((64, 8)),
            full((64, 24)),
            full((64, 24)),
            full((64, 24)),
            full((64, 24)),
            full((8, 32)),
            full((24, 32)),
        ],
        out_specs=pl.BlockSpec((BE, 32), lambda i: (i, 0)),
        out_shape=jax.ShapeDtypeStruct((E, 32), jnp.float32),
        interpret=interpret,
    )(edge_attr.T, g, W1 * 0.25, W2 * 0.125, jnp.asarray(dif),
      jnp.asarray(_Q24), jnp.asarray(pad128(_P24)), jnp.asarray(_RDS),
      jnp.asarray(pad128(_S64)), jnp.asarray(pad128(_CXYZ[0])),
      jnp.asarray(pad128(_CXYZ[1])), jnp.asarray(pad128(_CXYZ[2])),
      jnp.asarray(_R64S), jnp.asarray(_RBT),
      jnp.asarray(_RC[0]), jnp.asarray(_RC[1]), jnp.asarray(_RC[2]),
      jnp.asarray(_E8), jnp.asarray(_E24))


def _node_body(inv_sqrt_deg, xp_ref, a0_ref, a1_ref, oh_ref, u1_ref, u2_ref,
               s8_ref, t8_ref, si_ref, sj_ref, sij_ref, r512_ref,
               px_ref, py_ref, pz_ref, out_ref):
    h = _dot(oh_ref[...], u1_ref[...])
    w = _dot(_silu(h), u2_ref[...])
    wA = w[:, 0:512]
    wB = w[:, 512:1024]
    wC = w[:, 1024:1536]
    wD = w[:, 1536:2048]

    agg = (a0_ref[...] + a1_ref[...]) * inv_sqrt_deg
    ms = agg[:, 0:8]
    mvx = agg[:, 8:16]
    mvy = agg[:, 16:24]
    mvz = agg[:, 24:32]
    xp = xp_ref[...]
    xs = xp[:, 0:8]
    xvx = xp[:, 8:16]
    xvy = xp[:, 16:24]
    xvz = xp[:, 24:32]

    S8 = s8_ref[...]
    T8 = t8_ref[...]
    SI = si_ref[...]
    SJ = sj_ref[...]
    R512 = r512_ref[...]
    fan_inv = 1.0 / np.sqrt(2.0 * 64.0)

    dvv = (_dot(xvx, S8) * _dot(mvx, T8)
           + _dot(xvy, S8) * _dot(mvy, T8)
           + _dot(xvz, S8) * _dot(mvz, T8))
    XS = _dot(xs, SI)
    MS = _dot(ms, SJ)
    DV = _dot(dvv, sij_ref[...])

    out_s = _dot(XS * MS * wA + DV * wD * (1.0 / SQRT3), R512) * fan_inv
    ovx = _dot(XS * _dot(mvx, SJ) * wB + _dot(xvx, SI) * MS * wC, R512) * fan_inv
    ovy = _dot(XS * _dot(mvy, SJ) * wB + _dot(xvy, SI) * MS * wC, R512) * fan_inv
    ovz = _dot(XS * _dot(mvz, SJ) * wB + _dot(xvz, SI) * MS * wC, R512) * fan_inv
    out_v = (_dot(ovx, px_ref[...]) + _dot(ovy, py_ref[...])
             + _dot(ovz, pz_ref[...]))
    out_ref[...] = jnp.concatenate([out_s, out_v], axis=1)


def _node_call(xprep, a0, a1, onehot, U1, U2, inv_sqrt_deg, N, interpret=False):
    BN = 1000
    grid = (N // BN,)
    return pl.pallas_call(
        functools.partial(_node_body, inv_sqrt_deg),
        grid=grid,
        in_specs=[
            pl.BlockSpec((BN, 32), lambda i: (i, 0)),
            pl.BlockSpec((BN, 32), lambda i: (i, 0)),
            pl.BlockSpec((BN, 32), lambda i: (i, 0)),
            pl.BlockSpec((BN, 16), lambda i: (i, 0)),
            pl.BlockSpec((16, 64), lambda i: (0, 0)),
            pl.BlockSpec((64, 2048), lambda i: (0, 0)),
            pl.BlockSpec((8, 64), lambda i: (0, 0)),
            pl.BlockSpec((8, 64), lambda i: (0, 0)),
            pl.BlockSpec((8, 512), lambda i: (0, 0)),
            pl.BlockSpec((8, 512), lambda i: (0, 0)),
            pl.BlockSpec((64, 512), lambda i: (0, 0)),
            pl.BlockSpec((512, 8), lambda i: (0, 0)),
            pl.BlockSpec((8, 24), lambda i: (0, 0)),
            pl.BlockSpec((8, 24), lambda i: (0, 0)),
            pl.BlockSpec((8, 24), lambda i: (0, 0)),
        ],
        out_specs=pl.BlockSpec((BN, 32), lambda i: (i, 0)),
        out_shape=jax.ShapeDtypeStruct((N, 32), jnp.float32),
        interpret=interpret,
    )(xprep, a0, a1, onehot, U1 * 0.25, U2 * 0.125,
      jnp.asarray(_S8), jnp.asarray(_T8), jnp.asarray(_SI512),
      jnp.asarray(_SJ512), jnp.asarray(_SIJ512), jnp.asarray(_R512),
      jnp.asarray(_P[0]), jnp.asarray(_P[1]), jnp.asarray(_P[2]))


def _prep(x, pos):
    N = x.shape[0]
    xv = x[:, MUL:].reshape(N, MUL, 3)
    xprep = jnp.concatenate([x[:, :MUL], xv[:, :, 0], xv[:, :, 1], xv[:, :, 2]],
                            axis=1)
    table = jnp.concatenate([xprep, pos, jnp.zeros((N, 29), pos.dtype)],
                            axis=1)
    return xprep, table


def kernel(x, pos, edge_attr, onehot_node_features, W1, W2, U1, U2, edge_index):
    N = x.shape[0]
    E = edge_index.shape[1]
    xprep, table = _prep(x, pos)
    with jax.enable_x64():
        packed = (edge_index[0].astype(jnp.uint64)
                  | (edge_index[1].astype(jnp.uint64) << 32))
        idx2 = jax.lax.bitcast_convert_type(packed, jnp.int32)
    idx2d = idx2.reshape(2 * E // CS, CS)
    dst2d = edge_index[1].reshape(E // CS, CS)
    zeros_pad = jnp.zeros((NP, 32), jnp.float32)

    g2 = _gather_call(table, idx2d, E)
    msg = _edge_call(edge_attr, g2.reshape(E, 128), W1, W2, E)
    agg = _scatter_call(msg, dst2d, zeros_pad, E)
    inv_sqrt_deg = float(1.0 / np.sqrt(E / N))
    return _node_call(xprep, agg[:N], agg[NP:NP + N], onehot_node_features,
                      U1, U2, inv_sqrt_deg, N)
